# baseline (speedup 1.0000x reference)
import functools

import jax
import jax.numpy as jnp
from jax import lax
from jax.experimental import pallas as pl
from jax.experimental.pallas import tpu as pltpu
from jax.experimental.pallas import tpu_sc as plsc

_BATCH = 4
_SEQ = 2048
_EMBED = 64
_VOCAB = 100000
_FLAT = _BATCH * _SEQ

_INFO = plsc.get_sparse_core_info()
_NC = _INFO.num_cores
_NS = _INFO.num_subcores
_NW = _NC * _NS
_ROWS_W = _FLAT // _NW
_LANES = 16
_CHUNK = 128
_NWORD = _EMBED // 2
_HI = jnp.int32(-65536)


def _round_bf16(s):
    u = plsc.bitcast(s, jnp.int32)
    lsb = lax.bitwise_and(lax.shift_right_logical(u, 16), 1)
    return u + 0x7FFF + lsb


def _emb_body(tok_hbm, table_hbm, pos_hbm, out_hbm, idx_v, quad_v, dst_v,
              prow_v, out_v, sem):
    wid = lax.axis_index("s") * _NC + lax.axis_index("c")
    base = wid * _ROWS_W
    pltpu.sync_copy(tok_hbm.at[pl.ds(wid * 2, 2)], idx_v)
    for j in range(2):
        for q in range(_CHUNK // _LANES):
            sl = pl.ds(q * _LANES, _LANES)
            quad_v[j, sl] = lax.shift_right_logical(idx_v[j, sl], 2)
    copies = [
        pltpu.async_copy(table_hbm.at[quad_v.at[j]],
                         dst_v.at[pl.ds(j * _CHUNK, _CHUNK)], sem)
        for j in range(2)
    ]
    pbase = pl.multiple_of(lax.rem(base, _SEQ), _ROWS_W)
    pltpu.sync_copy(pos_hbm.at[pl.ds(pbase, _ROWS_W)], prow_v)
    for cp in copies:
        cp.wait()

    iota = lax.iota(jnp.int32, _LANES)
    ones = jnp.full((_LANES,), 1, jnp.int32)

    def group(g, carry):
        j = g // 8
        off = lax.rem(g, 8) * _LANES
        tokv = idx_v[j, pl.ds(off, _LANES)]
        colbase = lax.bitwise_and(tokv, 3) * _NWORD
        t_wrk = g * _LANES + iota
        for w in range(_NWORD):
            tw = plsc.load_gather(dst_v, [t_wrk, colbase + w])
            pw = plsc.load_gather(prow_v, [t_wrk, ones * w])
            alo = plsc.bitcast(lax.shift_left(tw, 16), jnp.float32)
            ahi = plsc.bitcast(lax.bitwise_and(tw, _HI), jnp.float32)
            plo = plsc.bitcast(lax.shift_left(pw, 16), jnp.float32)
            phi = plsc.bitcast(lax.bitwise_and(pw, _HI), jnp.float32)
            ulo = _round_bf16(alo + plo)
            uhi = _round_bf16(ahi + phi)
            word = lax.bitwise_or(lax.shift_right_logical(ulo, 16),
                                  lax.bitwise_and(uhi, _HI))
            plsc.store_scatter(out_v, [ones * w, t_wrk], word)
        return carry

    lax.fori_loop(0, _ROWS_W // _LANES, group, 0)
    bidx = base // _SEQ
    pltpu.sync_copy(out_v, out_hbm.at[bidx, :, pl.ds(pbase, _ROWS_W)])


_emb = functools.partial(
    pl.kernel,
    mesh=plsc.VectorSubcoreMesh(core_axis_name="c", subcore_axis_name="s"),
    out_type=jax.ShapeDtypeStruct((_BATCH, _NWORD, _SEQ), jnp.int32),
    scratch_types=[
        pltpu.VMEM((2, _CHUNK), jnp.int32),
        pltpu.VMEM((2, _CHUNK), jnp.int32),
        pltpu.VMEM((_ROWS_W, _CHUNK), jnp.int32),
        pltpu.VMEM((_ROWS_W, _NWORD), jnp.int32),
        pltpu.VMEM((_NWORD, _ROWS_W), jnp.int32),
        pltpu.SemaphoreType.DMA,
    ],
    compiler_params=pltpu.CompilerParams(use_tc_tiling_on_sc=False,
                                         needs_layout_passes=False),
)(_emb_body)


def kernel(tokens, token_table, pos_table):
    tok = tokens.astype(jnp.int32).reshape(_FLAT // _CHUNK, _CHUNK)
    tbl = lax.bitcast_convert_type(
        token_table.astype(jnp.bfloat16).reshape(_VOCAB // 4, _CHUNK, 2),
        jnp.int32)
    pos = lax.bitcast_convert_type(
        pos_table.astype(jnp.bfloat16).reshape(_SEQ, _NWORD, 2), jnp.int32)
    out = _emb(tok, tbl, pos)
    pairs = lax.bitcast_convert_type(out, jnp.bfloat16)
    return jnp.transpose(pairs, (0, 2, 1, 3)).reshape(_BATCH, _SEQ, _EMBED)


# speedup vs baseline: 27.4874x; 27.4874x over previous
import functools

import jax
import jax.numpy as jnp
from jax import lax
from jax.experimental import pallas as pl
from jax.experimental.pallas import tpu as pltpu
from jax.experimental.pallas import tpu_sc as plsc

_BATCH = 4
_SEQ = 2048
_EMBED = 64
_FLAT = _BATCH * _SEQ

_INFO = plsc.get_sparse_core_info()
_NC = _INFO.num_cores
_NS = _INFO.num_subcores
_NW = _NC * _NS
_ROWS_W = _FLAT // _NW
_CHUNK = 128
_NCHUNK = _ROWS_W // _CHUNK
_LANES = 16


def _emb_body(tok_hbm, table_hbm, pos_hbm, out_hbm, idx_v, trow_v, prow_v,
              out_v, sem):
    wid = lax.axis_index("s") * _NC + lax.axis_index("c")
    base = wid * _ROWS_W
    pltpu.sync_copy(tok_hbm.at[pl.ds(wid * _NCHUNK, _NCHUNK)], idx_v)
    copies = [
        pltpu.async_copy(table_hbm.at[idx_v.at[j]],
                         trow_v.at[pl.ds(j * _CHUNK, _CHUNK)], sem)
        for j in range(_NCHUNK)
    ]
    pbase = lax.rem(base, _SEQ)
    pltpu.sync_copy(pos_hbm.at[pl.ds(pbase, _ROWS_W)], prow_v)
    for cp in copies:
        cp.wait()

    def body(i, carry):
        for c in range(_EMBED // _LANES):
            sl = pl.ds(c * _LANES, _LANES)
            out_v[i, sl] = trow_v[i, sl] + prow_v[i, sl]
        return carry

    lax.fori_loop(0, _ROWS_W, body, 0)
    pltpu.sync_copy(out_v, out_hbm.at[pl.ds(base, _ROWS_W)])


_emb = functools.partial(
    pl.kernel,
    mesh=plsc.VectorSubcoreMesh(core_axis_name="c", subcore_axis_name="s"),
    out_type=jax.ShapeDtypeStruct((_FLAT, _EMBED), jnp.float32),
    scratch_types=[
        pltpu.VMEM((_NCHUNK, _CHUNK), jnp.int32),
        pltpu.VMEM((_ROWS_W, _EMBED), jnp.float32),
        pltpu.VMEM((_ROWS_W, _EMBED), jnp.float32),
        pltpu.VMEM((_ROWS_W, _EMBED), jnp.float32),
        pltpu.SemaphoreType.DMA,
    ],
    compiler_params=pltpu.CompilerParams(use_tc_tiling_on_sc=False),
)(_emb_body)


def kernel(tokens, token_table, pos_table):
    tok = tokens.astype(jnp.int32).reshape(_FLAT // _CHUNK, _CHUNK)
    out = _emb(tok, token_table, pos_table)
    return out.reshape(_BATCH, _SEQ, _EMBED).astype(jnp.bfloat16)
